# baseline (speedup 1.0000x reference)
import functools
import jax
import jax.numpy as jnp
from jax import lax
from jax.experimental import pallas as pl
from jax.experimental.pallas import tpu as pltpu, tpu_sc as plsc

_D = 1024
_C = 32
_UNROLL = 8
_CW = _C * _D


def sc_kernel(x, pe_table):
    B, S, _ = x.shape
    xf = x.reshape(B * S * _D)
    pef = pe_table.reshape(S * _D)
    NW = 32
    rows_per_w = S // NW
    n_chunks = rows_per_w // _C
    n_sl = _CW // (16 * _UNROLL)

    mesh = plsc.VectorSubcoreMesh(core_axis_name="c", subcore_axis_name="s")

    @functools.partial(
        pl.kernel,
        mesh=mesh,
        out_type=jax.ShapeDtypeStruct((B * S * _D,), jnp.float32),
        scratch_types=[
            pltpu.VMEM((_CW,), jnp.float32),
            pltpu.VMEM((_CW,), jnp.float32),
        ],
    )
    def k(xf_hbm, pe_hbm, out_hbm, pe_buf, x_buf):
        wid = lax.axis_index("s") * 2 + lax.axis_index("c")
        w_base = wid * rows_per_w * _D

        def chunk_body(ci, _):
            p0 = w_base + ci * _CW
            pltpu.sync_copy(pe_hbm.at[pl.ds(p0, _CW)], pe_buf)

            def batch_body(b, _):
                r0 = b * S * _D + p0
                pltpu.sync_copy(xf_hbm.at[pl.ds(r0, _CW)], x_buf)

                def add_body(i, _):
                    base = i * (16 * _UNROLL)
                    for u in range(_UNROLL):
                        sl = pl.ds(base + u * 16, 16)
                        x_buf[sl] = x_buf[sl] + pe_buf[sl]
                    return ()

                lax.fori_loop(0, n_sl, add_body, ())
                pltpu.sync_copy(x_buf, out_hbm.at[pl.ds(r0, _CW)])
                return ()

            lax.fori_loop(0, B, batch_body, ())
            return ()

        lax.fori_loop(0, n_chunks, chunk_body, ())

    out = k(xf, pef)
    return out.reshape(B, S, _D)


kernel = sc_kernel


# speedup vs baseline: 1.2243x; 1.2243x over previous
import functools
import jax
import jax.numpy as jnp
from jax import lax
from jax.experimental import pallas as pl
from jax.experimental.pallas import tpu as pltpu, tpu_sc as plsc

_D = 1024
_C = 16
_CW = _C * _D
_UNROLL = 8
_NSL = _CW // (16 * _UNROLL)


def sc_kernel(x, pe_table):
    B, S, _ = x.shape
    xf = x.reshape(B * S * _D)
    pef = pe_table.reshape(S * _D)
    NW = 32
    rows_per_w = S // NW
    n_chunks = rows_per_w // _C
    n_k = n_chunks // 2

    mesh = plsc.VectorSubcoreMesh(core_axis_name="c", subcore_axis_name="s")

    @functools.partial(
        pl.kernel,
        mesh=mesh,
        out_type=jax.ShapeDtypeStruct((B * S * _D,), jnp.float32),
        scratch_types=[
            pltpu.VMEM((_CW,), jnp.float32),
            pltpu.VMEM((_CW,), jnp.float32),
            pltpu.VMEM((_CW,), jnp.float32),
            pltpu.VMEM((_CW,), jnp.float32),
            pltpu.VMEM((_CW,), jnp.float32),
            pltpu.VMEM((_CW,), jnp.float32),
            pltpu.SemaphoreType.DMA,
            pltpu.SemaphoreType.DMA,
            pltpu.SemaphoreType.DMA,
            pltpu.SemaphoreType.DMA,
            pltpu.SemaphoreType.DMA,
            pltpu.SemaphoreType.DMA,
        ],
    )
    def k_fn(xf_hbm, pe_hbm, out_hbm, pe0, pe1, xi0, xi1, xo0, xo1,
             sp0, sp1, si0, si1, so0, so1):
        pe_bufs, pe_sems = (pe0, pe1), (sp0, sp1)
        xi_bufs, si_sems = (xi0, xi1), (si0, si1)
        xo_bufs, so_sems = (xo0, xo1), (so0, so1)

        wid = lax.axis_index("s") * 2 + lax.axis_index("c")
        w_base = wid * rows_per_w * _D

        def pe_off(ci):
            return w_base + ci * _CW

        def x_off(ci, b):
            return b * S * _D + w_base + ci * _CW

        def wait_in(xp):
            pltpu.make_async_copy(
                xf_hbm.at[pl.ds(0, _CW)], xi_bufs[xp], si_sems[xp]).wait()

        def wait_out(xp):
            pltpu.make_async_copy(
                xo_bufs[xp], out_hbm.at[pl.ds(0, _CW)], so_sems[xp]).wait()

        def wait_pe(cp):
            pltpu.make_async_copy(
                pe_hbm.at[pl.ds(0, _CW)], pe_bufs[cp], pe_sems[cp]).wait()

        pltpu.async_copy(pe_hbm.at[pl.ds(pe_off(0), _CW)], pe0, sp0)
        pltpu.async_copy(pe_hbm.at[pl.ds(pe_off(1), _CW)], pe1, sp1)
        pltpu.async_copy(xf_hbm.at[pl.ds(x_off(0, 0), _CW)], xi0, si0)
        pltpu.async_copy(xf_hbm.at[pl.ds(x_off(0, 1), _CW)], xi1, si1)

        def outer(k, _):
            for cpar in range(2):
                ci = 2 * k + cpar
                wait_pe(cpar)
                pe_buf = pe_bufs[cpar]
                for b in range(4):
                    xp = b % 2
                    xi, xo = xi_bufs[xp], xo_bufs[xp]
                    wait_in(xp)
                    if cpar == 0 and b < 2:
                        @pl.when(k > 0)
                        def _():
                            wait_out(xp)
                    else:
                        wait_out(xp)

                    def add_body(i, _):
                        base = i * (16 * _UNROLL)
                        for u2 in range(_UNROLL):
                            sl = pl.ds(base + u2 * 16, 16)
                            xo[sl] = xi[sl] + pe_buf[sl]
                        return ()

                    lax.fori_loop(0, _NSL, add_body, ())

                    if b < 2:
                        nci, nb = ci, b + 2
                        pltpu.async_copy(
                            xf_hbm.at[pl.ds(x_off(nci, nb), _CW)],
                            xi, si_sems[xp])
                    else:
                        nci, nb = ci + 1, b - 2
                        if cpar == 0:
                            pltpu.async_copy(
                                xf_hbm.at[pl.ds(x_off(nci, nb), _CW)],
                                xi, si_sems[xp])
                        else:
                            @pl.when(k < n_k - 1)
                            def _():
                                pltpu.async_copy(
                                    xf_hbm.at[pl.ds(x_off(nci, nb), _CW)],
                                    xi, si_sems[xp])

                    pltpu.async_copy(
                        xo, out_hbm.at[pl.ds(x_off(ci, b), _CW)], so_sems[xp])

                @pl.when(k < n_k - 1)
                def _():
                    pltpu.async_copy(
                        pe_hbm.at[pl.ds(pe_off(2 * k + cpar + 2), _CW)],
                        pe_bufs[cpar], pe_sems[cpar])
            return ()

        lax.fori_loop(0, n_k, outer, ())
        wait_out(0)
        wait_out(1)

    out = k_fn(xf, pef)
    return out.reshape(B, S, _D)


kernel = sc_kernel
